# Initial kernel scaffold; baseline (speedup 1.0000x reference)
import jax
import jax.numpy as jnp
from jax.experimental import pallas as pl


def kernel(x, coords):
    raise NotImplementedError("write your pallas kernel here")


# speedup vs baseline: 3.1797x; 3.1797x over previous
import functools

import jax
import jax.numpy as jnp
from jax import lax
from jax.experimental import pallas as pl
from jax.experimental.pallas import tpu as pltpu
from jax.experimental.pallas import tpu_sc as plsc

B, C, NX, NY = 4, 192, 224, 224
N = 16384
PLANE = NX * NY
NC, NS, L = 2, 16, 16
NW = NC * NS
WPB = NW // B
CPW = C // WPB

_mesh = plsc.VectorSubcoreMesh(
    core_axis_name="c", subcore_axis_name="s", num_cores=NC, num_subcores=NS
)


@functools.partial(
    pl.kernel,
    out_type=jax.ShapeDtypeStruct((B, C, N), jnp.float32),
    mesh=_mesh,
    scratch_types=[
        pltpu.VMEM((PLANE,), jnp.float32),
        pltpu.VMEM((N,), jnp.int32),
        pltpu.VMEM((N,), jnp.float32),
    ],
    compiler_params=pltpu.CompilerParams(needs_layout_passes=False),
)
def _grid_sampler(x_hbm, coords_hbm, out_hbm, plane_v, idx_v, row_v):
    wid = lax.axis_index("s") * NC + lax.axis_index("c")
    b = wid // WPB
    c0 = (wid % WPB) * CPW

    pltpu.sync_copy(coords_hbm.at[b], plane_v.at[pl.ds(0, 2 * N)])
    lanes = lax.iota(jnp.int32, L)

    def idx_body(i, _):
        base = i * (2 * L)
        yv = plsc.load_gather(plane_v, [base + lanes * 2])
        xv = plsc.load_gather(plane_v, [base + lanes * 2 + 1])
        px = jnp.clip(xv * (NX - 1), 0.0, float(NX)).astype(jnp.int32)
        py = jnp.clip(yv * (NY - 1), 0.0, float(NY)).astype(jnp.int32)
        px = jnp.minimum(px, NX - 1)
        py = jnp.minimum(py, NY - 1)
        idx_v[pl.ds(i * L, L)] = px * NY + py
        return 0

    lax.fori_loop(0, N // L, idx_body, 0, unroll=False)

    def plane_body(j, _):
        ci = c0 + j
        pltpu.sync_copy(x_hbm.at[b, ci], plane_v)

        def gather_body(i, _):
            iv = idx_v[pl.ds(i * L, L)]
            row_v[pl.ds(i * L, L)] = plsc.load_gather(plane_v, [iv])
            return 0

        lax.fori_loop(0, N // L, gather_body, 0, unroll=False)
        pltpu.sync_copy(row_v, out_hbm.at[b, ci])
        return 0

    lax.fori_loop(0, CPW, plane_body, 0, unroll=False)


def kernel(x, coords):
    x2 = x.reshape(B, C, PLANE)
    coords2 = coords.reshape(B, 2 * N)
    return _grid_sampler(x2, coords2)


# speedup vs baseline: 4.9163x; 1.5461x over previous
import functools

import jax
import jax.numpy as jnp
from jax import lax
from jax.experimental import pallas as pl
from jax.experimental.pallas import tpu as pltpu
from jax.experimental.pallas import tpu_sc as plsc

B, C, NX, NY = 4, 192, 224, 224
N = 16384
PLANE = NX * NY
NC, NS, L = 2, 16, 16
NW = NC * NS
WPB = NW // B
CPW = C // WPB
CH = 4096
NCHUNK = N // CH

_mesh = plsc.VectorSubcoreMesh(
    core_axis_name="c", subcore_axis_name="s", num_cores=NC, num_subcores=NS
)


@functools.partial(
    pl.kernel,
    out_type=jax.ShapeDtypeStruct((B, C, N), jnp.float32),
    mesh=_mesh,
    scratch_types=[
        pltpu.VMEM((PLANE,), jnp.float32),
        pltpu.VMEM((PLANE,), jnp.float32),
        pltpu.VMEM((N,), jnp.int32),
        pltpu.VMEM((CH,), jnp.float32),
        pltpu.VMEM((CH,), jnp.float32),
        pltpu.SemaphoreType.DMA,
        pltpu.SemaphoreType.DMA,
        pltpu.SemaphoreType.DMA,
    ],
    compiler_params=pltpu.CompilerParams(needs_layout_passes=False),
)
def _grid_sampler(
    x_hbm, coords_hbm, out_hbm, p0, p1, idx_v, ob0, ob1, psem, os0, os1
):
    wid = lax.axis_index("s") * NC + lax.axis_index("c")
    b = wid // WPB
    c0 = (wid % WPB) * CPW
    osems = (os0, os1)
    obufs = (ob0, ob1)

    pltpu.sync_copy(coords_hbm.at[b], p0.at[pl.ds(0, 2 * N)])
    lanes = lax.iota(jnp.int32, L)

    @plsc.parallel_loop(0, N // L, unroll=4)
    def _idx_body(i):
        base = i * (2 * L)
        yv = plsc.load_gather(p0, [base + lanes * 2])
        xv = plsc.load_gather(p0, [base + lanes * 2 + 1])
        px = jnp.clip(xv * (NX - 1), 0.0, float(NX)).astype(jnp.int32)
        py = jnp.clip(yv * (NY - 1), 0.0, float(NY)).astype(jnp.int32)
        px = jnp.minimum(px, NX - 1)
        py = jnp.minimum(py, NY - 1)
        idx_v[pl.ds(i * L, L)] = px * NY + py

    def _wait_plane():
        pltpu.make_async_copy(x_hbm.at[b, c0], p0, psem).wait()

    def _gather_plane(plane_v, ci, fired):
        for k in range(NCHUNK):
            par = k % 2
            obuf = obufs[par]
            dst = out_hbm.at[b, ci, pl.ds(k * CH, CH)]

            def _wait_chunk(p=par, d=dst):
                pltpu.make_async_copy(obufs[p], d, osems[p]).wait()

            if k >= 2:
                _wait_chunk()
            else:
                pl.when(fired + k >= 2)(_wait_chunk)

            @plsc.parallel_loop(0, CH // L, unroll=8)
            def _gather_body(i, k=k, obuf=obuf):
                iv = idx_v[pl.ds(k * CH + i * L, L)]
                obuf[pl.ds(i * L, L)] = plsc.load_gather(plane_v, [iv])

            pltpu.async_copy(obuf, dst, osems[par])
        return fired + NCHUNK

    pltpu.async_copy(x_hbm.at[b, c0], p0, psem)

    def plane_pair(jj, fired):
        j0 = 2 * jj
        _wait_plane()
        pltpu.async_copy(x_hbm.at[b, c0 + j0 + 1], p1, psem)
        fired = _gather_plane(p0, c0 + j0, fired)
        _wait_plane()
        @pl.when(jj < CPW // 2 - 1)
        def _prefetch_next():
            pltpu.async_copy(x_hbm.at[b, c0 + j0 + 2], p0, psem)
        fired = _gather_plane(p1, c0 + j0 + 1, fired)
        return fired

    lax.fori_loop(0, CPW // 2, plane_pair, 0, unroll=False)

    pltpu.make_async_copy(ob0, out_hbm.at[0, 0, pl.ds(2 * CH, CH)], os0).wait()
    pltpu.make_async_copy(ob1, out_hbm.at[0, 0, pl.ds(3 * CH, CH)], os1).wait()


def kernel(x, coords):
    x2 = x.reshape(B, C, PLANE)
    coords2 = coords.reshape(B, 2 * N)
    return _grid_sampler(x2, coords2)


# speedup vs baseline: 8.8281x; 1.7957x over previous
import functools

import jax
import jax.numpy as jnp
from jax import lax
from jax.experimental import pallas as pl
from jax.experimental.pallas import tpu as pltpu
from jax.experimental.pallas import tpu_sc as plsc

B, C, NX, NY = 4, 192, 224, 224
N = 16384
NC, NS, L = 2, 16, 16
NW = NC * NS
WPB = NW // B
CPW = C // WPB
CH = 512
NCHUNK = N // CH

_mesh = plsc.VectorSubcoreMesh(
    core_axis_name="c", subcore_axis_name="s", num_cores=NC, num_subcores=NS
)


@functools.partial(
    pl.kernel,
    out_type=jax.ShapeDtypeStruct((B, C, N), jnp.float32),
    mesh=_mesh,
    scratch_types=[
        pltpu.VMEM((NX, NY), jnp.float32),
        pltpu.VMEM((NX, NY), jnp.float32),
        pltpu.VMEM((N // 2,), jnp.int32),
        pltpu.VMEM((CH,), jnp.float32),
        pltpu.VMEM((CH,), jnp.float32),
        pltpu.SemaphoreType.DMA,
        pltpu.SemaphoreType.DMA,
        pltpu.SemaphoreType.DMA,
    ],
    compiler_params=pltpu.CompilerParams(needs_layout_passes=False),
)
def _grid_sampler(
    x_hbm, coords_hbm, out_hbm, p0, p1, idx_v, ob0, ob1, psem, os0, os1
):
    wid = lax.axis_index("s") * NC + lax.axis_index("c")
    b = wid // WPB
    c0 = (wid % WPB) * CPW
    lanes = lax.iota(jnp.int32, L)

    def _pack(xv, yv):
        px = jnp.clip(xv * (NX - 1), 0.0, float(NX)).astype(jnp.int32)
        py = jnp.clip(yv * (NY - 1), 0.0, float(NY)).astype(jnp.int32)
        px = jnp.minimum(px, NX - 1)
        py = jnp.minimum(py, NY - 1)
        return lax.shift_left(px, 8) | py

    def coords_chunk(ch, _):
        pltpu.sync_copy(coords_hbm.at[b, pl.ds(ch * CH, CH)], ob0)

        @plsc.parallel_loop(0, CH // (4 * L), unroll=4)
        def _idx_body(i):
            base = i * (4 * L)
            y0 = plsc.load_gather(ob0, [base + lanes * 4])
            x0 = plsc.load_gather(ob0, [base + lanes * 4 + 1])
            y1 = plsc.load_gather(ob0, [base + lanes * 4 + 2])
            x1 = plsc.load_gather(ob0, [base + lanes * 4 + 3])
            w = _pack(x0, y0) | lax.shift_left(_pack(x1, y1), 16)
            idx_v[pl.ds(ch * (CH // 4) + i * L, L)] = w

        return 0

    lax.fori_loop(0, (2 * N) // CH, coords_chunk, 0, unroll=False)

    def _wait_plane():
        pltpu.make_async_copy(x_hbm.at[b, c0], p0, psem).wait()

    def _gather_chunk(plane_v, obuf, s0):
        @plsc.parallel_loop(0, CH // (2 * L), unroll=8)
        def _gather_body(i):
            w = idx_v[pl.ds(s0 // 2 + i * L, L)]
            lo = w & 0xFFFF
            hi = lax.shift_right_logical(w, 16)
            v0 = plsc.load_gather(
                plane_v, [lax.shift_right_logical(lo, 8), lo & 255]
            )
            v1 = plsc.load_gather(
                plane_v, [lax.shift_right_logical(hi, 8), hi & 255]
            )
            pos = i * (2 * L) + lanes * 2
            plsc.store_scatter(obuf, [pos], v0)
            plsc.store_scatter(obuf, [pos + 1], v1)

    def _gather_plane(plane_v, ci, fired):
        def pair_body(m, fired):
            def _wait0():
                pltpu.make_async_copy(
                    ob0, out_hbm.at[0, 0, pl.ds(0, CH)], os0
                ).wait()

            def _wait1():
                pltpu.make_async_copy(
                    ob1, out_hbm.at[0, 0, pl.ds(0, CH)], os1
                ).wait()

            pl.when(fired >= 1)(_wait0)
            _gather_chunk(plane_v, ob0, (2 * m) * CH)
            pltpu.async_copy(ob0, out_hbm.at[b, ci, pl.ds(2 * m * CH, CH)], os0)
            pl.when(fired >= 1)(_wait1)
            _gather_chunk(plane_v, ob1, (2 * m + 1) * CH)
            pltpu.async_copy(
                ob1, out_hbm.at[b, ci, pl.ds((2 * m + 1) * CH, CH)], os1
            )
            return fired + 1

        return lax.fori_loop(0, NCHUNK // 2, pair_body, fired, unroll=False)

    pltpu.async_copy(x_hbm.at[b, c0], p0, psem)

    def plane_pair(jj, fired):
        j0 = 2 * jj
        _wait_plane()
        pltpu.async_copy(x_hbm.at[b, c0 + j0 + 1], p1, psem)
        fired = _gather_plane(p0, c0 + j0, fired)
        _wait_plane()

        @pl.when(jj < CPW // 2 - 1)
        def _prefetch_next():
            pltpu.async_copy(x_hbm.at[b, c0 + j0 + 2], p0, psem)

        fired = _gather_plane(p1, c0 + j0 + 1, fired)
        return fired

    lax.fori_loop(0, CPW // 2, plane_pair, 0, unroll=False)

    pltpu.make_async_copy(ob0, out_hbm.at[0, 0, pl.ds(0, CH)], os0).wait()
    pltpu.make_async_copy(ob1, out_hbm.at[0, 0, pl.ds(0, CH)], os1).wait()


def kernel(x, coords):
    coords2 = coords.reshape(B, 2 * N)
    return _grid_sampler(x, coords2)


# speedup vs baseline: 10.5409x; 1.1940x over previous
import functools

import jax
import jax.numpy as jnp
from jax import lax
from jax.experimental import pallas as pl
from jax.experimental.pallas import tpu as pltpu
from jax.experimental.pallas import tpu_sc as plsc

B, C, NX, NY = 4, 192, 224, 224
N = 16384
NC, NS, L = 2, 16, 16
NW = NC * NS
WPB = NW // B
CPW = C // WPB
CH = 512
NCHUNK = N // CH

_mesh = plsc.VectorSubcoreMesh(
    core_axis_name="c", subcore_axis_name="s", num_cores=NC, num_subcores=NS
)


@functools.partial(
    pl.kernel,
    out_type=jax.ShapeDtypeStruct((B, C, N), jnp.float32),
    mesh=_mesh,
    scratch_types=[
        pltpu.VMEM((NX, NY), jnp.float32),
        pltpu.VMEM((NX, NY), jnp.float32),
        pltpu.VMEM((N // 2,), jnp.int32),
        pltpu.VMEM((CH,), jnp.float32),
        pltpu.VMEM((CH,), jnp.float32),
        pltpu.SemaphoreType.DMA,
        pltpu.SemaphoreType.DMA,
        pltpu.SemaphoreType.DMA,
    ],
    compiler_params=pltpu.CompilerParams(needs_layout_passes=False),
)
def _grid_sampler(
    x_hbm, coords_hbm, out_hbm, p0, p1, idx_v, ob0, ob1, psem, os0, os1
):
    wid = lax.axis_index("s") * NC + lax.axis_index("c")
    b = wid // WPB
    c0 = (wid % WPB) * CPW
    lanes = lax.iota(jnp.int32, L)

    def _pack(xv, yv):
        px = jnp.clip(xv * (NX - 1), 0.0, float(NX)).astype(jnp.int32)
        py = jnp.clip(yv * (NY - 1), 0.0, float(NY)).astype(jnp.int32)
        px = jnp.minimum(px, NX - 1)
        py = jnp.minimum(py, NY - 1)
        return lax.shift_left(px, 8) | py

    def coords_chunk(ch, _):
        pltpu.sync_copy(coords_hbm.at[b, 0, pl.ds(ch * CH, CH)], ob0)
        pltpu.sync_copy(coords_hbm.at[b, 1, pl.ds(ch * CH, CH)], ob1)

        @plsc.parallel_loop(0, CH // (2 * L), unroll=4)
        def _idx_body(i):
            base = i * (2 * L)
            y0 = plsc.load_gather(ob0, [base + lanes * 2])
            y1 = plsc.load_gather(ob0, [base + lanes * 2 + 1])
            x0 = plsc.load_gather(ob1, [base + lanes * 2])
            x1 = plsc.load_gather(ob1, [base + lanes * 2 + 1])
            w = _pack(x0, y0) | lax.shift_left(_pack(x1, y1), 16)
            idx_v[pl.ds(ch * (CH // 2) + i * L, L)] = w

        return 0

    lax.fori_loop(0, N // CH, coords_chunk, 0, unroll=False)

    def _wait_plane():
        pltpu.make_async_copy(x_hbm.at[b, c0], p0, psem).wait()

    def _gather_chunk(plane_v, obuf, s0):
        @plsc.parallel_loop(0, CH // (2 * L), unroll=8)
        def _gather_body(i):
            w = idx_v[pl.ds(s0 // 2 + i * L, L)]
            lo = w & 0xFFFF
            hi = lax.shift_right_logical(w, 16)
            v0 = plsc.load_gather(
                plane_v, [lax.shift_right_logical(lo, 8), lo & 255]
            )
            v1 = plsc.load_gather(
                plane_v, [lax.shift_right_logical(hi, 8), hi & 255]
            )
            pos = i * (2 * L) + lanes * 2
            plsc.store_scatter(obuf, [pos], v0)
            plsc.store_scatter(obuf, [pos + 1], v1)

    def _gather_plane(plane_v, ci, fired):
        def pair_body(m, fired):
            def _wait0():
                pltpu.make_async_copy(
                    ob0, out_hbm.at[0, 0, pl.ds(0, CH)], os0
                ).wait()

            def _wait1():
                pltpu.make_async_copy(
                    ob1, out_hbm.at[0, 0, pl.ds(0, CH)], os1
                ).wait()

            pl.when(fired >= 1)(_wait0)
            _gather_chunk(plane_v, ob0, (2 * m) * CH)
            pltpu.async_copy(ob0, out_hbm.at[b, ci, pl.ds(2 * m * CH, CH)], os0)
            pl.when(fired >= 1)(_wait1)
            _gather_chunk(plane_v, ob1, (2 * m + 1) * CH)
            pltpu.async_copy(
                ob1, out_hbm.at[b, ci, pl.ds((2 * m + 1) * CH, CH)], os1
            )
            return fired + 1

        return lax.fori_loop(0, NCHUNK // 2, pair_body, fired, unroll=False)

    pltpu.async_copy(x_hbm.at[b, c0], p0, psem)

    def plane_pair(jj, fired):
        j0 = 2 * jj
        _wait_plane()
        pltpu.async_copy(x_hbm.at[b, c0 + j0 + 1], p1, psem)
        fired = _gather_plane(p0, c0 + j0, fired)
        _wait_plane()

        @pl.when(jj < CPW // 2 - 1)
        def _prefetch_next():
            pltpu.async_copy(x_hbm.at[b, c0 + j0 + 2], p0, psem)

        fired = _gather_plane(p1, c0 + j0 + 1, fired)
        return fired

    lax.fori_loop(0, CPW // 2, plane_pair, 0, unroll=False)

    pltpu.make_async_copy(ob0, out_hbm.at[0, 0, pl.ds(0, CH)], os0).wait()
    pltpu.make_async_copy(ob1, out_hbm.at[0, 0, pl.ds(0, CH)], os1).wait()


def kernel(x, coords):
    coords_t = coords.transpose(0, 2, 1)
    return _grid_sampler(x, coords_t)


# speedup vs baseline: 10.5774x; 1.0035x over previous
import functools

import jax
import jax.numpy as jnp
from jax import lax
from jax.experimental import pallas as pl
from jax.experimental.pallas import tpu as pltpu
from jax.experimental.pallas import tpu_sc as plsc

B, C, NX, NY = 4, 192, 224, 224
N = 16384
NC, NS, L = 2, 16, 16
NW = NC * NS
WPB = NW // B
CPW = C // WPB
CH = 512
NCHUNK = N // CH

_mesh = plsc.VectorSubcoreMesh(
    core_axis_name="c", subcore_axis_name="s", num_cores=NC, num_subcores=NS
)


@functools.partial(
    pl.kernel,
    out_type=jax.ShapeDtypeStruct((B, C, N), jnp.float32),
    mesh=_mesh,
    scratch_types=[
        pltpu.VMEM((NX, NY), jnp.float32),
        pltpu.VMEM((NX, NY), jnp.float32),
        pltpu.VMEM((N // 2,), jnp.int32),
        pltpu.VMEM((CH,), jnp.float32),
        pltpu.VMEM((CH,), jnp.float32),
        pltpu.SemaphoreType.DMA,
        pltpu.SemaphoreType.DMA,
        pltpu.SemaphoreType.DMA,
    ],
    compiler_params=pltpu.CompilerParams(needs_layout_passes=False),
)
def _grid_sampler(
    x_hbm, coords_hbm, out_hbm, p0, p1, idx_v, ob0, ob1, psem, os0, os1
):
    wid = lax.axis_index("s") * NC + lax.axis_index("c")
    b = wid // WPB
    c0 = (wid % WPB) * CPW
    lanes = lax.iota(jnp.int32, L)

    def _pack(xv, yv):
        px = jnp.clip(xv * (NX - 1), 0.0, float(NX)).astype(jnp.int32)
        py = jnp.clip(yv * (NY - 1), 0.0, float(NY)).astype(jnp.int32)
        px = jnp.minimum(px, NX - 1)
        py = jnp.minimum(py, NY - 1)
        return lax.shift_left(px, 8) | py

    def coords_chunk(ch, _):
        pltpu.sync_copy(coords_hbm.at[b, 0, pl.ds(ch * CH, CH)], ob0)
        pltpu.sync_copy(coords_hbm.at[b, 1, pl.ds(ch * CH, CH)], ob1)

        @plsc.parallel_loop(0, CH // (2 * L), unroll=4)
        def _idx_body(i):
            y0 = ob0[pl.ds(i * L, L)]
            y1 = ob0[pl.ds(CH // 2 + i * L, L)]
            x0 = ob1[pl.ds(i * L, L)]
            x1 = ob1[pl.ds(CH // 2 + i * L, L)]
            w = _pack(x0, y0) | lax.shift_left(_pack(x1, y1), 16)
            idx_v[pl.ds(ch * (CH // 2) + i * L, L)] = w

        return 0

    lax.fori_loop(0, N // CH, coords_chunk, 0, unroll=False)

    def _wait_plane():
        pltpu.make_async_copy(x_hbm.at[b, c0], p0, psem).wait()

    def _gather_chunk(plane_v, obuf, w0):
        @plsc.parallel_loop(0, CH // (2 * L), unroll=8)
        def _gather_body(i):
            w = idx_v[pl.ds(w0 + i * L, L)]
            lo = w & 0xFFFF
            hi = lax.shift_right_logical(w, 16)
            v0 = plsc.load_gather(
                plane_v, [lax.shift_right_logical(lo, 8), lo & 255]
            )
            v1 = plsc.load_gather(
                plane_v, [lax.shift_right_logical(hi, 8), hi & 255]
            )
            obuf[pl.ds(i * L, L)] = v0
            obuf[pl.ds(CH // 2 + i * L, L)] = v1

    def _gather_plane(plane_v, ci, fired):
        def pair_body(m, fired):
            def _wait0():
                pltpu.make_async_copy(
                    ob0, out_hbm.at[0, 0, pl.ds(0, CH)], os0
                ).wait()

            def _wait1():
                pltpu.make_async_copy(
                    ob1, out_hbm.at[0, 0, pl.ds(0, CH)], os1
                ).wait()

            pl.when(fired >= 1)(_wait0)
            _gather_chunk(plane_v, ob0, (2 * m) * (CH // 2))
            pltpu.async_copy(ob0, out_hbm.at[b, ci, pl.ds(2 * m * CH, CH)], os0)
            pl.when(fired >= 1)(_wait1)
            _gather_chunk(plane_v, ob1, (2 * m + 1) * (CH // 2))
            pltpu.async_copy(
                ob1, out_hbm.at[b, ci, pl.ds((2 * m + 1) * CH, CH)], os1
            )
            return fired + 1

        return lax.fori_loop(0, NCHUNK // 2, pair_body, fired, unroll=False)

    pltpu.async_copy(x_hbm.at[b, c0], p0, psem)

    def plane_pair(jj, fired):
        j0 = 2 * jj
        _wait_plane()
        pltpu.async_copy(x_hbm.at[b, c0 + j0 + 1], p1, psem)
        fired = _gather_plane(p0, c0 + j0, fired)
        _wait_plane()

        @pl.when(jj < CPW // 2 - 1)
        def _prefetch_next():
            pltpu.async_copy(x_hbm.at[b, c0 + j0 + 2], p0, psem)

        fired = _gather_plane(p1, c0 + j0 + 1, fired)
        return fired

    lax.fori_loop(0, CPW // 2, plane_pair, 0, unroll=False)

    pltpu.make_async_copy(ob0, out_hbm.at[0, 0, pl.ds(0, CH)], os0).wait()
    pltpu.make_async_copy(ob1, out_hbm.at[0, 0, pl.ds(0, CH)], os1).wait()


def kernel(x, coords):
    coords_t = coords.transpose(0, 2, 1)
    return _grid_sampler(x, coords_t)


# speedup vs baseline: 10.7446x; 1.0158x over previous
import functools

import jax
import jax.numpy as jnp
from jax import lax
from jax.experimental import pallas as pl
from jax.experimental.pallas import tpu as pltpu
from jax.experimental.pallas import tpu_sc as plsc

B, C, NX, NY = 4, 192, 224, 224
N = 16384
NC, NS, L = 2, 16, 16
NW = NC * NS
WPB = NW // B
CPW = C // WPB
CH = 512
NCHUNK = N // CH

_mesh = plsc.VectorSubcoreMesh(
    core_axis_name="c", subcore_axis_name="s", num_cores=NC, num_subcores=NS
)


@functools.partial(
    pl.kernel,
    out_type=jax.ShapeDtypeStruct((B, C, N), jnp.float32),
    mesh=_mesh,
    scratch_types=[
        pltpu.VMEM((NX, NY), jnp.float32),
        pltpu.VMEM((NX, NY), jnp.float32),
        pltpu.VMEM((N // 2,), jnp.int32),
        pltpu.VMEM((CH,), jnp.float32),
        pltpu.VMEM((CH,), jnp.float32),
        pltpu.SemaphoreType.DMA,
        pltpu.SemaphoreType.DMA,
        pltpu.SemaphoreType.DMA,
        pltpu.SemaphoreType.DMA,
    ],
    compiler_params=pltpu.CompilerParams(needs_layout_passes=False),
)
def _grid_sampler(
    x_hbm, coords_hbm, out_hbm, p0, p1, idx_v, ob0, ob1, ps0, ps1, os0, os1
):
    wid = lax.axis_index("s") * NC + lax.axis_index("c")
    b = wid // WPB
    c0 = (wid % WPB) * CPW
    lanes = lax.iota(jnp.int32, L)

    def _pack(xv, yv):
        px = jnp.clip(xv * (NX - 1), 0.0, float(NX)).astype(jnp.int32)
        py = jnp.clip(yv * (NY - 1), 0.0, float(NY)).astype(jnp.int32)
        px = jnp.minimum(px, NX - 1)
        py = jnp.minimum(py, NY - 1)
        return lax.shift_left(px, 8) | py

    pltpu.async_copy(x_hbm.at[b, c0], p0, ps0)
    pltpu.async_copy(x_hbm.at[b, c0 + 1], p1, ps1)

    def coords_chunk(ch, _):
        pltpu.sync_copy(coords_hbm.at[b, 0, pl.ds(ch * CH, CH)], ob0)
        pltpu.sync_copy(coords_hbm.at[b, 1, pl.ds(ch * CH, CH)], ob1)

        @plsc.parallel_loop(0, CH // (2 * L), unroll=4)
        def _idx_body(i):
            y0 = ob0[pl.ds(i * L, L)]
            y1 = ob0[pl.ds(CH // 2 + i * L, L)]
            x0 = ob1[pl.ds(i * L, L)]
            x1 = ob1[pl.ds(CH // 2 + i * L, L)]
            w = _pack(x0, y0) | lax.shift_left(_pack(x1, y1), 16)
            idx_v[pl.ds(ch * (CH // 2) + i * L, L)] = w

        return 0

    lax.fori_loop(0, N // CH, coords_chunk, 0, unroll=False)

    def _gather_chunk(plane_v, obuf, w0):
        @plsc.parallel_loop(0, CH // (2 * L), unroll=8)
        def _gather_body(i):
            w = idx_v[pl.ds(w0 + i * L, L)]
            lo = w & 0xFFFF
            hi = lax.shift_right_logical(w, 16)
            v0 = plsc.load_gather(
                plane_v, [lax.shift_right_logical(lo, 8), lo & 255]
            )
            v1 = plsc.load_gather(
                plane_v, [lax.shift_right_logical(hi, 8), hi & 255]
            )
            obuf[pl.ds(i * L, L)] = v0
            obuf[pl.ds(CH // 2 + i * L, L)] = v1

    def _gather_plane(plane_v, ci, fired):
        def pair_body(m, fired):
            def _wait0():
                pltpu.make_async_copy(
                    ob0, out_hbm.at[0, 0, pl.ds(0, CH)], os0
                ).wait()

            def _wait1():
                pltpu.make_async_copy(
                    ob1, out_hbm.at[0, 0, pl.ds(0, CH)], os1
                ).wait()

            pl.when(fired >= 1)(_wait0)
            _gather_chunk(plane_v, ob0, (2 * m) * (CH // 2))
            pltpu.async_copy(ob0, out_hbm.at[b, ci, pl.ds(2 * m * CH, CH)], os0)
            pl.when(fired >= 1)(_wait1)
            _gather_chunk(plane_v, ob1, (2 * m + 1) * (CH // 2))
            pltpu.async_copy(
                ob1, out_hbm.at[b, ci, pl.ds((2 * m + 1) * CH, CH)], os1
            )
            return fired + 1

        return lax.fori_loop(0, NCHUNK // 2, pair_body, fired, unroll=False)

    def plane_pair(jj, fired):
        j0 = 2 * jj
        pltpu.make_async_copy(x_hbm.at[b, c0], p0, ps0).wait()
        fired = _gather_plane(p0, c0 + j0, fired)

        @pl.when(jj < CPW // 2 - 1)
        def _prefetch_p0():
            pltpu.async_copy(x_hbm.at[b, c0 + j0 + 2], p0, ps0)

        pltpu.make_async_copy(x_hbm.at[b, c0], p1, ps1).wait()
        fired = _gather_plane(p1, c0 + j0 + 1, fired)

        @pl.when(jj < CPW // 2 - 1)
        def _prefetch_p1():
            pltpu.async_copy(x_hbm.at[b, c0 + j0 + 3], p1, ps1)

        return fired

    lax.fori_loop(0, CPW // 2, plane_pair, 0, unroll=False)

    pltpu.make_async_copy(ob0, out_hbm.at[0, 0, pl.ds(0, CH)], os0).wait()
    pltpu.make_async_copy(ob1, out_hbm.at[0, 0, pl.ds(0, CH)], os1).wait()


def kernel(x, coords):
    coords_t = coords.transpose(0, 2, 1)
    return _grid_sampler(x, coords_t)


# speedup vs baseline: 10.8368x; 1.0086x over previous
import functools

import jax
import jax.numpy as jnp
from jax import lax
from jax.experimental import pallas as pl
from jax.experimental.pallas import tpu as pltpu
from jax.experimental.pallas import tpu_sc as plsc

B, C, NX, NY = 4, 192, 224, 224
N = 16384
NC, NS, L = 2, 16, 16
NW = NC * NS
WPB = NW // B
CPW = C // WPB
CH = 512
NCHUNK = N // CH

_mesh = plsc.VectorSubcoreMesh(
    core_axis_name="c", subcore_axis_name="s", num_cores=NC, num_subcores=NS
)


@functools.partial(
    pl.kernel,
    out_type=jax.ShapeDtypeStruct((B, C, N), jnp.float32),
    mesh=_mesh,
    scratch_types=[
        pltpu.VMEM((NX, NY), jnp.float32),
        pltpu.VMEM((NX, NY), jnp.float32),
        pltpu.VMEM((N // 2,), jnp.int32),
        pltpu.VMEM((CH,), jnp.float32),
        pltpu.VMEM((CH,), jnp.float32),
        pltpu.SemaphoreType.DMA,
        pltpu.SemaphoreType.DMA,
        pltpu.SemaphoreType.DMA,
        pltpu.SemaphoreType.DMA,
    ],
    compiler_params=pltpu.CompilerParams(needs_layout_passes=False),
)
def _grid_sampler(
    x_hbm, coords_hbm, out_hbm, p0, p1, idx_v, ob0, ob1, ps0, ps1, os0, os1
):
    wid = lax.axis_index("s") * NC + lax.axis_index("c")
    b = wid // WPB
    c0 = (wid % WPB) * CPW
    lanes = lax.iota(jnp.int32, L)

    def _pack(xv, yv):
        px = jnp.clip(xv * (NX - 1), 0.0, float(NX)).astype(jnp.int32)
        py = jnp.clip(yv * (NY - 1), 0.0, float(NY)).astype(jnp.int32)
        px = jnp.minimum(px, NX - 1)
        py = jnp.minimum(py, NY - 1)
        return lax.shift_left(px, 8) | py

    pltpu.async_copy(x_hbm.at[b, c0], p0, ps0)
    pltpu.async_copy(x_hbm.at[b, c0 + 1], p1, ps1)

    def coords_chunk(ch, _):
        pltpu.sync_copy(coords_hbm.at[b, 0, pl.ds(ch * CH, CH)], ob0)
        pltpu.sync_copy(coords_hbm.at[b, 1, pl.ds(ch * CH, CH)], ob1)

        @plsc.parallel_loop(0, CH // (2 * L), unroll=2)
        def _idx_body(i):
            y0 = ob0[pl.ds(i * L, L)]
            y1 = ob0[pl.ds(CH // 2 + i * L, L)]
            x0 = ob1[pl.ds(i * L, L)]
            x1 = ob1[pl.ds(CH // 2 + i * L, L)]
            w = _pack(x0, y0) | lax.shift_left(_pack(x1, y1), 16)
            idx_v[pl.ds(ch * (CH // 2) + i * L, L)] = w

        return 0

    lax.fori_loop(0, N // CH, coords_chunk, 0, unroll=False)

    def _gather_chunk(plane_v, obuf, w0):
        @plsc.parallel_loop(0, CH // (2 * L), unroll=4)
        def _gather_body(i):
            w = idx_v[pl.ds(w0 + i * L, L)]
            lo = w & 0xFFFF
            hi = lax.shift_right_logical(w, 16)
            v0 = plsc.load_gather(
                plane_v, [lax.shift_right_logical(lo, 8), lo & 255]
            )
            v1 = plsc.load_gather(
                plane_v, [lax.shift_right_logical(hi, 8), hi & 255]
            )
            obuf[pl.ds(i * L, L)] = v0
            obuf[pl.ds(CH // 2 + i * L, L)] = v1

    def _gather_plane(plane_v, ci, fired):
        def pair_body(m, fired):
            def _wait0():
                pltpu.make_async_copy(
                    ob0, out_hbm.at[0, 0, pl.ds(0, CH)], os0
                ).wait()

            def _wait1():
                pltpu.make_async_copy(
                    ob1, out_hbm.at[0, 0, pl.ds(0, CH)], os1
                ).wait()

            pl.when(fired >= 1)(_wait0)
            _gather_chunk(plane_v, ob0, (2 * m) * (CH // 2))
            pltpu.async_copy(ob0, out_hbm.at[b, ci, pl.ds(2 * m * CH, CH)], os0)
            pl.when(fired >= 1)(_wait1)
            _gather_chunk(plane_v, ob1, (2 * m + 1) * (CH // 2))
            pltpu.async_copy(
                ob1, out_hbm.at[b, ci, pl.ds((2 * m + 1) * CH, CH)], os1
            )
            return fired + 1

        return lax.fori_loop(0, NCHUNK // 2, pair_body, fired, unroll=False)

    def plane_pair(jj, fired):
        j0 = 2 * jj
        pltpu.make_async_copy(x_hbm.at[b, c0], p0, ps0).wait()
        fired = _gather_plane(p0, c0 + j0, fired)

        @pl.when(jj < CPW // 2 - 1)
        def _prefetch_p0():
            pltpu.async_copy(x_hbm.at[b, c0 + j0 + 2], p0, ps0)

        pltpu.make_async_copy(x_hbm.at[b, c0], p1, ps1).wait()
        fired = _gather_plane(p1, c0 + j0 + 1, fired)

        @pl.when(jj < CPW // 2 - 1)
        def _prefetch_p1():
            pltpu.async_copy(x_hbm.at[b, c0 + j0 + 3], p1, ps1)

        return fired

    lax.fori_loop(0, CPW // 2, plane_pair, 0, unroll=False)

    pltpu.make_async_copy(ob0, out_hbm.at[0, 0, pl.ds(0, CH)], os0).wait()
    pltpu.make_async_copy(ob1, out_hbm.at[0, 0, pl.ds(0, CH)], os1).wait()


def kernel(x, coords):
    coords_t = coords.transpose(0, 2, 1)
    return _grid_sampler(x, coords_t)


# speedup vs baseline: 12.2350x; 1.1290x over previous
import functools

import jax
import jax.numpy as jnp
from jax import lax
from jax.experimental import pallas as pl
from jax.experimental.pallas import tpu as pltpu
from jax.experimental.pallas import tpu_sc as plsc

B, C, NX, NY = 4, 192, 224, 224
N = 16384
NC, NS, L = 2, 16, 16
NW = NC * NS
WPB = NW // B
CPW = C // WPB
CH = 1024
NCHUNK = N // CH

_mesh = plsc.VectorSubcoreMesh(
    core_axis_name="c", subcore_axis_name="s", num_cores=NC, num_subcores=NS
)


@functools.partial(
    pl.kernel,
    out_type=jax.ShapeDtypeStruct((B, C, N), jnp.float32),
    mesh=_mesh,
    scratch_types=[
        pltpu.VMEM((NX, NY), jnp.float32),
        pltpu.VMEM((NX, NY), jnp.float32),
        pltpu.VMEM((N // 2,), jnp.int32),
        pltpu.VMEM((CH,), jnp.float32),
        pltpu.VMEM((CH,), jnp.float32),
        pltpu.SemaphoreType.DMA,
        pltpu.SemaphoreType.DMA,
        pltpu.SemaphoreType.DMA,
        pltpu.SemaphoreType.DMA,
    ],
    compiler_params=pltpu.CompilerParams(needs_layout_passes=False),
)
def _grid_sampler(
    x_hbm, coords_hbm, out_hbm, p0, p1, idx_v, ob0, ob1, ps0, ps1, os0, os1
):
    wid = lax.axis_index("s") * NC + lax.axis_index("c")
    b = wid // WPB
    c0 = (wid % WPB) * CPW
    lanes = lax.iota(jnp.int32, L)

    def _pack(xv, yv):
        px = jnp.clip(xv * (NX - 1), 0.0, float(NX)).astype(jnp.int32)
        py = jnp.clip(yv * (NY - 1), 0.0, float(NY)).astype(jnp.int32)
        px = jnp.minimum(px, NX - 1)
        py = jnp.minimum(py, NY - 1)
        return lax.shift_left(px, 8) | py

    pltpu.async_copy(x_hbm.at[b, c0], p0, ps0)
    pltpu.async_copy(x_hbm.at[b, c0 + 1], p1, ps1)

    def coords_chunk(ch, _):
        pltpu.sync_copy(coords_hbm.at[b, 0, pl.ds(ch * CH, CH)], ob0)
        pltpu.sync_copy(coords_hbm.at[b, 1, pl.ds(ch * CH, CH)], ob1)

        @plsc.parallel_loop(0, CH // (2 * L), unroll=2)
        def _idx_body(i):
            y0 = ob0[pl.ds(i * L, L)]
            y1 = ob0[pl.ds(CH // 2 + i * L, L)]
            x0 = ob1[pl.ds(i * L, L)]
            x1 = ob1[pl.ds(CH // 2 + i * L, L)]
            w = _pack(x0, y0) | lax.shift_left(_pack(x1, y1), 16)
            idx_v[pl.ds(ch * (CH // 2) + i * L, L)] = w

        return 0

    lax.fori_loop(0, N // CH, coords_chunk, 0, unroll=False)

    def _gather_chunk(plane_v, obuf, w0):
        @plsc.parallel_loop(0, CH // (2 * L), unroll=4)
        def _gather_body(i):
            w = idx_v[pl.ds(w0 + i * L, L)]
            lo = w & 0xFFFF
            hi = lax.shift_right_logical(w, 16)
            v0 = plsc.load_gather(
                plane_v, [lax.shift_right_logical(lo, 8), lo & 255]
            )
            v1 = plsc.load_gather(
                plane_v, [lax.shift_right_logical(hi, 8), hi & 255]
            )
            obuf[pl.ds(i * L, L)] = v0
            obuf[pl.ds(CH // 2 + i * L, L)] = v1

    def _gather_plane(plane_v, ci, fired):
        def pair_body(m, fired):
            def _wait0():
                pltpu.make_async_copy(
                    ob0, out_hbm.at[0, 0, pl.ds(0, CH)], os0
                ).wait()

            def _wait1():
                pltpu.make_async_copy(
                    ob1, out_hbm.at[0, 0, pl.ds(0, CH)], os1
                ).wait()

            pl.when(fired >= 1)(_wait0)
            _gather_chunk(plane_v, ob0, (2 * m) * (CH // 2))
            pltpu.async_copy(ob0, out_hbm.at[b, ci, pl.ds(2 * m * CH, CH)], os0)
            pl.when(fired >= 1)(_wait1)
            _gather_chunk(plane_v, ob1, (2 * m + 1) * (CH // 2))
            pltpu.async_copy(
                ob1, out_hbm.at[b, ci, pl.ds((2 * m + 1) * CH, CH)], os1
            )
            return fired + 1

        return lax.fori_loop(0, NCHUNK // 2, pair_body, fired, unroll=False)

    def plane_pair(jj, fired):
        j0 = 2 * jj
        pltpu.make_async_copy(x_hbm.at[b, c0], p0, ps0).wait()
        fired = _gather_plane(p0, c0 + j0, fired)

        @pl.when(jj < CPW // 2 - 1)
        def _prefetch_p0():
            pltpu.async_copy(x_hbm.at[b, c0 + j0 + 2], p0, ps0)

        pltpu.make_async_copy(x_hbm.at[b, c0], p1, ps1).wait()
        fired = _gather_plane(p1, c0 + j0 + 1, fired)

        @pl.when(jj < CPW // 2 - 1)
        def _prefetch_p1():
            pltpu.async_copy(x_hbm.at[b, c0 + j0 + 3], p1, ps1)

        return fired

    lax.fori_loop(0, CPW // 2, plane_pair, 0, unroll=False)

    pltpu.make_async_copy(ob0, out_hbm.at[0, 0, pl.ds(0, CH)], os0).wait()
    pltpu.make_async_copy(ob1, out_hbm.at[0, 0, pl.ds(0, CH)], os1).wait()


def kernel(x, coords):
    coords_t = coords.transpose(0, 2, 1)
    return _grid_sampler(x, coords_t)


# speedup vs baseline: 12.2771x; 1.0034x over previous
import functools

import jax
import jax.numpy as jnp
from jax import lax
from jax.experimental import pallas as pl
from jax.experimental.pallas import tpu as pltpu
from jax.experimental.pallas import tpu_sc as plsc

B, C, NX, NY = 4, 192, 224, 224
N = 16384
NC, NS, L = 2, 16, 16
NW = NC * NS
WPB = NW // B
CPW = C // WPB
CH = 1024
NCHUNK = N // CH

_mesh = plsc.VectorSubcoreMesh(
    core_axis_name="c", subcore_axis_name="s", num_cores=NC, num_subcores=NS
)


@functools.partial(
    pl.kernel,
    out_type=jax.ShapeDtypeStruct((B, C, N), jnp.float32),
    mesh=_mesh,
    scratch_types=[
        pltpu.VMEM((NX, NY), jnp.float32),
        pltpu.VMEM((NX, NY), jnp.float32),
        pltpu.VMEM((N // 2,), jnp.int32),
        pltpu.VMEM((CH,), jnp.float32),
        pltpu.VMEM((CH,), jnp.float32),
        pltpu.SemaphoreType.DMA,
        pltpu.SemaphoreType.DMA,
        pltpu.SemaphoreType.DMA,
        pltpu.SemaphoreType.DMA,
    ],
    compiler_params=pltpu.CompilerParams(needs_layout_passes=False),
)
def _grid_sampler(
    x_hbm, coords_hbm, out_hbm, p0, p1, idx_v, ob0, ob1, ps0, ps1, os0, os1
):
    wid = lax.axis_index("s") * NC + lax.axis_index("c")
    b = wid // WPB
    c0 = (wid % WPB) * CPW
    lanes = lax.iota(jnp.int32, L)

    def _pack(xv, yv):
        px = jnp.clip(xv * (NX - 1), 0.0, float(NX)).astype(jnp.int32)
        py = jnp.clip(yv * (NY - 1), 0.0, float(NY)).astype(jnp.int32)
        px = jnp.minimum(px, NX - 1)
        py = jnp.minimum(py, NY - 1)
        return lax.shift_left(px, 8) | py

    pltpu.async_copy(x_hbm.at[b, c0], p0, ps0)
    pltpu.async_copy(x_hbm.at[b, c0 + 1], p1, ps1)

    def coords_chunk(ch, _):
        pltpu.sync_copy(coords_hbm.at[b, 0, pl.ds(ch * CH, CH)], ob0)
        pltpu.sync_copy(coords_hbm.at[b, 1, pl.ds(ch * CH, CH)], ob1)

        @plsc.parallel_loop(0, CH // (2 * L), unroll=2)
        def _idx_body(i):
            y0 = ob0[pl.ds(i * L, L)]
            y1 = ob0[pl.ds(CH // 2 + i * L, L)]
            x0 = ob1[pl.ds(i * L, L)]
            x1 = ob1[pl.ds(CH // 2 + i * L, L)]
            w = _pack(x0, y0) | lax.shift_left(_pack(x1, y1), 16)
            idx_v[pl.ds(ch * (CH // 2) + i * L, L)] = w

        return 0

    lax.fori_loop(0, N // CH, coords_chunk, 0, unroll=False)

    def _gather_chunk(plane_v, obuf, w0):
        @plsc.parallel_loop(0, CH // (2 * L), unroll=2)
        def _gather_body(i):
            w = idx_v[pl.ds(w0 + i * L, L)]
            lo = w & 0xFFFF
            hi = lax.shift_right_logical(w, 16)
            v0 = plsc.load_gather(
                plane_v, [lax.shift_right_logical(lo, 8), lo & 255]
            )
            v1 = plsc.load_gather(
                plane_v, [lax.shift_right_logical(hi, 8), hi & 255]
            )
            obuf[pl.ds(i * L, L)] = v0
            obuf[pl.ds(CH // 2 + i * L, L)] = v1

    def _gather_plane(plane_v, ci, fired):
        def pair_body(m, fired):
            def _wait0():
                pltpu.make_async_copy(
                    ob0, out_hbm.at[0, 0, pl.ds(0, CH)], os0
                ).wait()

            def _wait1():
                pltpu.make_async_copy(
                    ob1, out_hbm.at[0, 0, pl.ds(0, CH)], os1
                ).wait()

            pl.when(fired >= 1)(_wait0)
            _gather_chunk(plane_v, ob0, (2 * m) * (CH // 2))
            pltpu.async_copy(ob0, out_hbm.at[b, ci, pl.ds(2 * m * CH, CH)], os0)
            pl.when(fired >= 1)(_wait1)
            _gather_chunk(plane_v, ob1, (2 * m + 1) * (CH // 2))
            pltpu.async_copy(
                ob1, out_hbm.at[b, ci, pl.ds((2 * m + 1) * CH, CH)], os1
            )
            return fired + 1

        return lax.fori_loop(0, NCHUNK // 2, pair_body, fired, unroll=False)

    def plane_pair(jj, fired):
        j0 = 2 * jj
        pltpu.make_async_copy(x_hbm.at[b, c0], p0, ps0).wait()
        fired = _gather_plane(p0, c0 + j0, fired)

        @pl.when(jj < CPW // 2 - 1)
        def _prefetch_p0():
            pltpu.async_copy(x_hbm.at[b, c0 + j0 + 2], p0, ps0)

        pltpu.make_async_copy(x_hbm.at[b, c0], p1, ps1).wait()
        fired = _gather_plane(p1, c0 + j0 + 1, fired)

        @pl.when(jj < CPW // 2 - 1)
        def _prefetch_p1():
            pltpu.async_copy(x_hbm.at[b, c0 + j0 + 3], p1, ps1)

        return fired

    lax.fori_loop(0, CPW // 2, plane_pair, 0, unroll=False)

    pltpu.make_async_copy(ob0, out_hbm.at[0, 0, pl.ds(0, CH)], os0).wait()
    pltpu.make_async_copy(ob1, out_hbm.at[0, 0, pl.ds(0, CH)], os1).wait()


def kernel(x, coords):
    coords_t = coords.transpose(0, 2, 1)
    return _grid_sampler(x, coords_t)
